# Initial kernel scaffold; baseline (speedup 1.0000x reference)
import jax
import jax.numpy as jnp
from jax.experimental import pallas as pl


def kernel(pij, noise_full, noise_i, noise_j, logits_weight, topic_w, topic_b, i, j):
    raise NotImplementedError("write your pallas kernel here")


# speedup vs baseline: 1.2597x; 1.2597x over previous
import jax
import jax.numpy as jnp
from jax.experimental import pallas as pl
from jax.experimental.pallas import tpu as pltpu

_EPS = 1e-9
_NOBS = 8192
_NT = 128
_ND = 64
_B = 8192

_G1 = 8
_BM1 = _NOBS // _G1

_G2 = 32
_BM2 = _NOBS // _G2
_BB2 = _B // _G2
_JT = 256
_NJT = _NOBS // _JT


def _gumbel(u):
    return -jnp.log(-jnp.log(u + _EPS) + _EPS)


def _softmax(l):
    m = jnp.max(l, axis=-1, keepdims=True)
    e = jnp.exp(l - m)
    return e / jnp.sum(e, axis=-1, keepdims=True)


def _prep_body(logits_ref, noise_ref, w_ref, b_ref, x_ref, sq_ref):
    z = _softmax(logits_ref[...] + _gumbel(noise_ref[...]))
    x = jax.lax.dot_general(z, w_ref[...], (((1,), (1,)), ((), ())),
                            preferred_element_type=jnp.float32) + b_ref[...]
    x_ref[...] = x
    sq_ref[...] = jnp.sum(x * x, axis=-1, keepdims=True)


def _main_body(xl_ref, xr_ref, lw_ref, ni_ref, nj_ref, ii_ref, jj_ref, tw_ref,
               part_ref, den_ref, ti_ref, tj_ref):
    gbase = pl.program_id(0) * _BB2
    iota8 = jax.lax.broadcasted_iota(jnp.int32, (8, _NT), 0)
    for (idx_ref, dst) in ((ii_ref, ti_ref), (jj_ref, tj_ref)):
        for grp in range(_BB2 // 8):
            blk = jnp.zeros((8, _NT), jnp.float32)
            for u in range(8):
                idx = idx_ref[gbase + grp * 8 + u]
                start = pl.multiple_of((idx >> 3) << 3, 8)
                chunk = lw_ref[pl.ds(start, 8), :]
                amt = ((u + 8) - (idx & 7)) & 7
                rolled = pltpu.roll(chunk, amt, axis=0)
                blk = jnp.where(iota8 == u, rolled, blk)
            dst[grp * 8:(grp + 1) * 8, :] = blk
    zi = _softmax(ti_ref[...] + _gumbel(ni_ref[...]))
    zj = _softmax(tj_ref[...] + _gumbel(nj_ref[...]))
    dxt = jax.lax.dot_general(tw_ref[...], zi - zj, (((1,), (1,)), ((), ())),
                              preferred_element_type=jnp.float32)
    den = jnp.float32(_ND) + jnp.sum(dxt * dxt, axis=0, keepdims=True)
    den_ref[...] = den.reshape(1, 1, _BB2)
    xlb = xl_ref[...]
    acc = jnp.zeros((8, 128), jnp.float32)
    for t in range(_NJT):
        xrb = xr_ref[t * _JT:(t + 1) * _JT, :]
        tt = jax.lax.dot_general(xlb, xrb, (((1,), (1,)), ((), ())),
                                 preferred_element_type=jnp.float32)
        r = 1.0 / tt
        v = r[:, :128] + r[:, 128:]
        while v.shape[0] > 8:
            h = v.shape[0] // 2
            v = v[:h, :] + v[h:, :]
        acc = acc + v
    part_ref[...] = acc


def _final_body(pij_ref, den_ref, ps_ref, out_ref):
    ps = jnp.sum(ps_ref[...], axis=0, keepdims=True)
    tot = jnp.sum(ps, axis=1, keepdims=True)
    logpart = jnp.log(tot - jnp.float32(_NOBS))
    pij = pij_ref[...]
    out_ref[...] = pij * (jnp.log(pij) + jnp.log(den_ref[...]) + logpart)


def kernel(pij, noise_full, noise_i, noise_j, logits_weight, topic_w, topic_b, i, j):
    f32 = jnp.float32
    wpad = jnp.concatenate([topic_w, jnp.zeros((_NT - _ND, _NT), f32)], axis=0)
    bpad = jnp.concatenate([topic_b, jnp.zeros((_NT - _ND,), f32)]).reshape(1, _NT)

    x, sq = pl.pallas_call(
        _prep_body,
        grid=(_G1,),
        in_specs=[
            pl.BlockSpec((_BM1, _NT), lambda g: (g, 0)),
            pl.BlockSpec((_BM1, _NT), lambda g: (g, 0)),
            pl.BlockSpec((_NT, _NT), lambda g: (0, 0)),
            pl.BlockSpec((1, _NT), lambda g: (0, 0)),
        ],
        out_specs=[
            pl.BlockSpec((_BM1, _NT), lambda g: (g, 0)),
            pl.BlockSpec((_BM1, 1), lambda g: (g, 0)),
        ],
        out_shape=[
            jax.ShapeDtypeStruct((_NOBS, _NT), f32),
            jax.ShapeDtypeStruct((_NOBS, 1), f32),
        ],
        compiler_params=pltpu.CompilerParams(
            dimension_semantics=("parallel",)),
        name="topic_prep",
    )(logits_weight, noise_full, wpad, bpad)

    ones = jnp.ones((_NOBS, 1), f32)
    zer = jnp.zeros((_NOBS, _NT - _ND - 2), f32)
    xl = jnp.concatenate([-2.0 * x[:, :_ND], 1.0 + sq, ones, zer],
                         axis=1).astype(jnp.bfloat16)
    xr = jnp.concatenate([x[:, :_ND], ones, sq, zer],
                         axis=1).astype(jnp.bfloat16)

    parts, den = pl.pallas_call(
        _main_body,
        grid=(_G2,),
        in_specs=[
            pl.BlockSpec((_BM2, _NT), lambda g: (g, 0)),
            pl.BlockSpec((_NOBS, _NT), lambda g: (0, 0)),
            pl.BlockSpec((_NOBS, _NT), lambda g: (0, 0)),
            pl.BlockSpec((_BB2, _NT), lambda g: (g, 0)),
            pl.BlockSpec((_BB2, _NT), lambda g: (g, 0)),
            pl.BlockSpec(memory_space=pltpu.SMEM),
            pl.BlockSpec(memory_space=pltpu.SMEM),
            pl.BlockSpec((_ND, _NT), lambda g: (0, 0)),
        ],
        out_specs=[
            pl.BlockSpec((8, 128), lambda g: (g, 0)),
            pl.BlockSpec((1, 1, _BB2), lambda g: (g, 0, 0)),
        ],
        out_shape=[
            jax.ShapeDtypeStruct((_G2 * 8, 128), f32),
            jax.ShapeDtypeStruct((_G2, 1, _BB2), f32),
        ],
        scratch_shapes=[
            pltpu.VMEM((_BB2, _NT), f32),
            pltpu.VMEM((_BB2, _NT), f32),
        ],
        compiler_params=pltpu.CompilerParams(
            dimension_semantics=("parallel",)),
        name="topic_main",
    )(xl, xr, logits_weight, noise_i, noise_j,
      i.astype(jnp.int32), j.astype(jnp.int32), topic_w)

    loss = pl.pallas_call(
        _final_body,
        out_shape=jax.ShapeDtypeStruct((1, _B), f32),
        name="topic_final",
    )(pij.reshape(1, _B), den.reshape(1, _B), parts)
    return loss.reshape(_B)


# speedup vs baseline: 1.8276x; 1.4509x over previous
import jax
import jax.numpy as jnp
from jax.experimental import pallas as pl
from jax.experimental.pallas import tpu as pltpu

_EPS = 1e-9
_NOBS = 8192
_NT = 128
_ND = 64
_B = 8192

_G1 = 8
_BM1 = _NOBS // _G1

_G2 = 32
_BM2 = _NOBS // _G2
_BB2 = _B // _G2
_JT = 256
_NK = _G2 // 2 + 1


def _gumbel(u):
    return -jnp.log(-jnp.log(u + _EPS) + _EPS)


def _softmax(l):
    m = jnp.max(l, axis=-1, keepdims=True)
    e = jnp.exp(l - m)
    return e / jnp.sum(e, axis=-1, keepdims=True)


def _prep_body(logits_ref, noise_ref, w_ref, b_ref, xl_ref, xr_ref):
    z = _softmax(logits_ref[...] + _gumbel(noise_ref[...]))
    x = jax.lax.dot_general(z, w_ref[...], (((1,), (1,)), ((), ())),
                            preferred_element_type=jnp.float32) + b_ref[...]
    sq = jnp.sum(x * x, axis=-1, keepdims=True)
    lane = jax.lax.broadcasted_iota(jnp.int32, (_BM1, _NT), 1)
    e64 = lane == _ND
    e65 = lane == _ND + 1
    aug_l = jnp.where(e64, 1.0 + sq, jnp.where(e65, 1.0, 0.0))
    aug_r = jnp.where(e64, 1.0, jnp.where(e65, sq, 0.0))
    xl_ref[...] = (aug_l - 2.0 * x).astype(jnp.bfloat16)
    xr_ref[...] = (x + aug_r).astype(jnp.bfloat16)


def _main_body(xl_ref, xr_ref, lw_ref, ni_ref, nj_ref,
               sti_ref, ami_ref, stj_ref, amj_ref, tw_ref,
               part_ref, den_ref, ti_ref, tj_ref):
    gid = pl.program_id(0)
    gbase = gid * _BB2
    for (st_ref, am_ref, dst) in ((sti_ref, ami_ref, ti_ref),
                                  (stj_ref, amj_ref, tj_ref)):
        for r in range(_BB2):
            st = pl.multiple_of(st_ref[gbase + r], 8)
            chunk = lw_ref[pl.ds(st, 8), :]
            rolled = pltpu.roll(chunk, am_ref[gbase + r], axis=0)
            dst[r:r + 1, :] = rolled[0:1, :]
    zi = _softmax(ti_ref[...] + _gumbel(ni_ref[...]))
    zj = _softmax(tj_ref[...] + _gumbel(nj_ref[...]))
    dxt = jax.lax.dot_general(tw_ref[...], zi - zj, (((1,), (1,)), ((), ())),
                              preferred_element_type=jnp.float32)
    den = jnp.float32(_ND) + jnp.sum(dxt * dxt, axis=0, keepdims=True)
    den_ref[...] = den.reshape(1, 1, _BB2)
    xlb = xl_ref[...]
    acc1 = jnp.zeros((8, 128), jnp.float32)
    acc2 = jnp.zeros((8, 128), jnp.float32)
    for k in range(_NK):
        jo = pl.multiple_of(((gid + k) & (_G2 - 1)) * _JT, _JT)
        xrb = xr_ref[pl.ds(jo, _JT), :]
        tt = jax.lax.dot_general(xlb, xrb, (((1,), (1,)), ((), ())),
                                 preferred_element_type=jnp.float32)
        rr = 1.0 / tt
        v = rr[:, :128] + rr[:, 128:]
        while v.shape[0] > 8:
            h = v.shape[0] // 2
            v = v[:h, :] + v[h:, :]
        if k == 0 or k == _NK - 1:
            acc1 = acc1 + v
        else:
            acc2 = acc2 + v
    part_ref[...] = acc1 + 2.0 * acc2


def _final_body(pij_ref, den_ref, ps_ref, out_ref):
    ps = jnp.sum(ps_ref[...], axis=0, keepdims=True)
    tot = jnp.sum(ps, axis=1, keepdims=True)
    logpart = jnp.log(tot - jnp.float32(_NOBS))
    pij = pij_ref[...]
    out_ref[...] = pij * (jnp.log(pij) + jnp.log(den_ref[...]) + logpart)


def kernel(pij, noise_full, noise_i, noise_j, logits_weight, topic_w, topic_b, i, j):
    f32 = jnp.float32
    wpad = jnp.concatenate([topic_w, jnp.zeros((_NT - _ND, _NT), f32)], axis=0)
    bpad = jnp.concatenate([topic_b, jnp.zeros((_NT - _ND,), f32)]).reshape(1, _NT)
    i32 = i.astype(jnp.int32)
    j32 = j.astype(jnp.int32)
    sti = (i32 >> 3) << 3
    ami = (-i32) & 7
    stj = (j32 >> 3) << 3
    amj = (-j32) & 7

    xl, xr = pl.pallas_call(
        _prep_body,
        grid=(_G1,),
        in_specs=[
            pl.BlockSpec((_BM1, _NT), lambda g: (g, 0)),
            pl.BlockSpec((_BM1, _NT), lambda g: (g, 0)),
            pl.BlockSpec((_NT, _NT), lambda g: (0, 0)),
            pl.BlockSpec((1, _NT), lambda g: (0, 0)),
        ],
        out_specs=[
            pl.BlockSpec((_BM1, _NT), lambda g: (g, 0)),
            pl.BlockSpec((_BM1, _NT), lambda g: (g, 0)),
        ],
        out_shape=[
            jax.ShapeDtypeStruct((_NOBS, _NT), jnp.bfloat16),
            jax.ShapeDtypeStruct((_NOBS, _NT), jnp.bfloat16),
        ],
        compiler_params=pltpu.CompilerParams(
            dimension_semantics=("parallel",)),
        name="topic_prep",
    )(logits_weight, noise_full, wpad, bpad)

    parts, den = pl.pallas_call(
        _main_body,
        grid=(_G2,),
        in_specs=[
            pl.BlockSpec((_BM2, _NT), lambda g: (g, 0)),
            pl.BlockSpec((_NOBS, _NT), lambda g: (0, 0)),
            pl.BlockSpec((_NOBS, _NT), lambda g: (0, 0)),
            pl.BlockSpec((_BB2, _NT), lambda g: (g, 0)),
            pl.BlockSpec((_BB2, _NT), lambda g: (g, 0)),
            pl.BlockSpec(memory_space=pltpu.SMEM),
            pl.BlockSpec(memory_space=pltpu.SMEM),
            pl.BlockSpec(memory_space=pltpu.SMEM),
            pl.BlockSpec(memory_space=pltpu.SMEM),
            pl.BlockSpec((_ND, _NT), lambda g: (0, 0)),
        ],
        out_specs=[
            pl.BlockSpec((8, 128), lambda g: (g, 0)),
            pl.BlockSpec((1, 1, _BB2), lambda g: (g, 0, 0)),
        ],
        out_shape=[
            jax.ShapeDtypeStruct((_G2 * 8, 128), f32),
            jax.ShapeDtypeStruct((_G2, 1, _BB2), f32),
        ],
        scratch_shapes=[
            pltpu.VMEM((_BB2, _NT), f32),
            pltpu.VMEM((_BB2, _NT), f32),
        ],
        compiler_params=pltpu.CompilerParams(
            dimension_semantics=("parallel",)),
        name="topic_main",
    )(xl, xr, logits_weight, noise_i, noise_j,
      sti, ami, stj, amj, topic_w)

    loss = pl.pallas_call(
        _final_body,
        out_shape=jax.ShapeDtypeStruct((1, _B), f32),
        name="topic_final",
    )(pij.reshape(1, _B), den.reshape(1, _B), parts)
    return loss.reshape(_B)


# speedup vs baseline: 1.9409x; 1.0620x over previous
import jax
import jax.numpy as jnp
from jax.experimental import pallas as pl
from jax.experimental.pallas import tpu as pltpu

_EPS = 1e-9
_NOBS = 8192
_NT = 128
_ND = 64
_B = 8192

_G1 = 8
_BM1 = _NOBS // _G1

_G2 = 32
_BM2 = _NOBS // _G2
_BB2 = _B // _G2
_JT = 256
_NK = _G2 // 2 + 1


def _gumbel(u):
    return -jnp.log(-jnp.log(u + _EPS) + _EPS)


def _softmax(l):
    m = jnp.max(l, axis=-1, keepdims=True)
    e = jnp.exp(l - m)
    return e / jnp.sum(e, axis=-1, keepdims=True)


def _prep_body(logits_ref, noise_ref, w_ref, b_ref, xl_ref, xr_ref):
    z = _softmax(logits_ref[...] + _gumbel(noise_ref[...]))
    x = jax.lax.dot_general(z, w_ref[...], (((1,), (1,)), ((), ())),
                            preferred_element_type=jnp.float32) + b_ref[...]
    sq = jnp.sum(x * x, axis=-1, keepdims=True)
    lane = jax.lax.broadcasted_iota(jnp.int32, (_BM1, _NT), 1)
    e64 = lane == _ND
    e65 = lane == _ND + 1
    aug_l = jnp.where(e64, 1.0 + sq, jnp.where(e65, 1.0, 0.0))
    aug_r = jnp.where(e64, 1.0, jnp.where(e65, sq, 0.0))
    xl_ref[...] = (aug_l - 2.0 * x).astype(jnp.bfloat16)
    xr_ref[...] = (x + aug_r).astype(jnp.bfloat16)


def _main_body(xl_ref, xr_ref, lw_ref, ni_ref, nj_ref,
               sti_ref, ami_ref, stj_ref, amj_ref, tw_ref,
               part_ref, den_ref, ti_ref, tj_ref):
    gid = pl.program_id(0)
    gbase = gid * _BB2
    nrows = 2 * _BB2

    def _gather(q):
        st_ref, am_ref, dst = ((sti_ref, ami_ref, ti_ref) if q < _BB2
                               else (stj_ref, amj_ref, tj_ref))
        r = q % _BB2
        st = pl.multiple_of(st_ref[gbase + r], 8)
        chunk = lw_ref[pl.ds(st, 8), :]
        rolled = pltpu.roll(chunk, am_ref[gbase + r], axis=0)
        dst[r:r + 1, :] = rolled[0:1, :]

    gpt = -(-nrows // _NK)
    xlb = xl_ref[...]
    acc1 = jnp.zeros((8, 128), jnp.float32)
    acc2 = jnp.zeros((8, 128), jnp.float32)
    for k in range(_NK):
        jo = pl.multiple_of(((gid + k) & (_G2 - 1)) * _JT, _JT)
        xrb = xr_ref[pl.ds(jo, _JT), :]
        tt = jax.lax.dot_general(xlb, xrb, (((1,), (1,)), ((), ())),
                                 preferred_element_type=jnp.float32)
        for q in range(k * gpt, min((k + 1) * gpt, nrows)):
            _gather(q)
        rr = 1.0 / tt
        v = rr[:, :128] + rr[:, 128:]
        while v.shape[0] > 8:
            h = v.shape[0] // 2
            v = v[:h, :] + v[h:, :]
        if k == 0 or k == _NK - 1:
            acc1 = acc1 + v
        else:
            acc2 = acc2 + v
    part_ref[...] = acc1 + 2.0 * acc2
    zi = _softmax(ti_ref[...] + _gumbel(ni_ref[...]))
    zj = _softmax(tj_ref[...] + _gumbel(nj_ref[...]))
    dxt = jax.lax.dot_general(tw_ref[...], zi - zj, (((1,), (1,)), ((), ())),
                              preferred_element_type=jnp.float32)
    den = jnp.float32(_ND) + jnp.sum(dxt * dxt, axis=0, keepdims=True)
    den_ref[...] = den.reshape(1, 1, _BB2)


def _final_body(pij_ref, den_ref, ps_ref, out_ref):
    ps = jnp.sum(ps_ref[...], axis=0, keepdims=True)
    tot = jnp.sum(ps, axis=1, keepdims=True)
    logpart = jnp.log(tot - jnp.float32(_NOBS))
    pij = pij_ref[...]
    out_ref[...] = pij * (jnp.log(pij) + jnp.log(den_ref[...]) + logpart)


def kernel(pij, noise_full, noise_i, noise_j, logits_weight, topic_w, topic_b, i, j):
    f32 = jnp.float32
    wpad = jnp.concatenate([topic_w, jnp.zeros((_NT - _ND, _NT), f32)], axis=0)
    bpad = jnp.concatenate([topic_b, jnp.zeros((_NT - _ND,), f32)]).reshape(1, _NT)
    i32 = i.astype(jnp.int32)
    j32 = j.astype(jnp.int32)
    sti = (i32 >> 3) << 3
    ami = (-i32) & 7
    stj = (j32 >> 3) << 3
    amj = (-j32) & 7

    xl, xr = pl.pallas_call(
        _prep_body,
        grid=(_G1,),
        in_specs=[
            pl.BlockSpec((_BM1, _NT), lambda g: (g, 0)),
            pl.BlockSpec((_BM1, _NT), lambda g: (g, 0)),
            pl.BlockSpec((_NT, _NT), lambda g: (0, 0)),
            pl.BlockSpec((1, _NT), lambda g: (0, 0)),
        ],
        out_specs=[
            pl.BlockSpec((_BM1, _NT), lambda g: (g, 0)),
            pl.BlockSpec((_BM1, _NT), lambda g: (g, 0)),
        ],
        out_shape=[
            jax.ShapeDtypeStruct((_NOBS, _NT), jnp.bfloat16),
            jax.ShapeDtypeStruct((_NOBS, _NT), jnp.bfloat16),
        ],
        compiler_params=pltpu.CompilerParams(
            dimension_semantics=("parallel",)),
        name="topic_prep",
    )(logits_weight, noise_full, wpad, bpad)

    parts, den = pl.pallas_call(
        _main_body,
        grid=(_G2,),
        in_specs=[
            pl.BlockSpec((_BM2, _NT), lambda g: (g, 0)),
            pl.BlockSpec((_NOBS, _NT), lambda g: (0, 0)),
            pl.BlockSpec((_NOBS, _NT), lambda g: (0, 0)),
            pl.BlockSpec((_BB2, _NT), lambda g: (g, 0)),
            pl.BlockSpec((_BB2, _NT), lambda g: (g, 0)),
            pl.BlockSpec(memory_space=pltpu.SMEM),
            pl.BlockSpec(memory_space=pltpu.SMEM),
            pl.BlockSpec(memory_space=pltpu.SMEM),
            pl.BlockSpec(memory_space=pltpu.SMEM),
            pl.BlockSpec((_ND, _NT), lambda g: (0, 0)),
        ],
        out_specs=[
            pl.BlockSpec((8, 128), lambda g: (g, 0)),
            pl.BlockSpec((1, 1, _BB2), lambda g: (g, 0, 0)),
        ],
        out_shape=[
            jax.ShapeDtypeStruct((_G2 * 8, 128), f32),
            jax.ShapeDtypeStruct((_G2, 1, _BB2), f32),
        ],
        scratch_shapes=[
            pltpu.VMEM((_BB2, _NT), f32),
            pltpu.VMEM((_BB2, _NT), f32),
        ],
        compiler_params=pltpu.CompilerParams(
            dimension_semantics=("parallel",)),
        name="topic_main",
    )(xl, xr, logits_weight, noise_i, noise_j,
      sti, ami, stj, amj, topic_w)

    loss = pl.pallas_call(
        _final_body,
        out_shape=jax.ShapeDtypeStruct((1, _B), f32),
        name="topic_final",
    )(pij.reshape(1, _B), den.reshape(1, _B), parts)
    return loss.reshape(_B)


# speedup vs baseline: 1.9703x; 1.0151x over previous
import jax
import jax.numpy as jnp
from jax.experimental import pallas as pl
from jax.experimental.pallas import tpu as pltpu

_EPS = 1e-9
_NOBS = 8192
_NT = 128
_ND = 64
_B = 8192

_G1 = 8
_BM1 = _NOBS // _G1

_G2 = 32
_BM2 = _NOBS // _G2
_BB2 = _B // _G2
_JT = 256
_NK = _G2 // 2 + 1


def _gumbel(u):
    return -jnp.log(-jnp.log(u + _EPS) + _EPS)


def _softmax(l):
    m = jnp.max(l, axis=-1, keepdims=True)
    e = jnp.exp(l - m)
    return e / jnp.sum(e, axis=-1, keepdims=True)


def _prep_body(logits_ref, noise_ref, w_ref, b_ref, xl_ref, xr_ref):
    z = _softmax(logits_ref[...] + _gumbel(noise_ref[...]))
    wb = w_ref[...] + b_ref[...]
    wbp = jnp.concatenate([wb, jnp.zeros((_NT - _ND, _NT), jnp.float32)],
                          axis=0)
    x = jax.lax.dot_general(z, wbp, (((1,), (1,)), ((), ())),
                            preferred_element_type=jnp.float32)
    sq = jnp.sum(x * x, axis=-1, keepdims=True)
    lane = jax.lax.broadcasted_iota(jnp.int32, (_BM1, _NT), 1)
    e64 = lane == _ND
    e65 = lane == _ND + 1
    aug_l = jnp.where(e64, 1.0 + sq, jnp.where(e65, 1.0, 0.0))
    aug_r = jnp.where(e64, 1.0, jnp.where(e65, sq, 0.0))
    xl_ref[...] = (aug_l - 2.0 * x).astype(jnp.bfloat16)
    xr_ref[...] = (x + aug_r).astype(jnp.bfloat16)


def _main_body(xl_ref, xr_ref, lw_ref, ni_ref, nj_ref, ii_ref, jj_ref,
               tw_ref, pij_ref, out_ref, acc_ref, den_ref, ti_ref, tj_ref):
    gid = pl.program_id(0)

    @pl.when(gid < _G2)
    def _work():
        gbase = gid * _BB2
        nrows = 2 * _BB2

        def _gather(q):
            idx_ref, dst = ((ii_ref, ti_ref) if q < _BB2
                            else (jj_ref, tj_ref))
            idx = idx_ref[gbase + (q % _BB2)]
            st = pl.multiple_of((idx >> 3) << 3, 8)
            chunk = lw_ref[pl.ds(st, 8), :]
            rolled = pltpu.roll(chunk, (0 - idx) & 7, axis=0)
            dst[(q % _BB2):(q % _BB2) + 1, :] = rolled[0:1, :]

        gpt = -(-nrows // _NK)
        xlb = xl_ref[...]
        acc1 = jnp.zeros((8, 128), jnp.float32)
        acc2 = jnp.zeros((8, 128), jnp.float32)
        for k in range(_NK):
            jo = pl.multiple_of(((gid + k) & (_G2 - 1)) * _JT, _JT)
            xrb = xr_ref[pl.ds(jo, _JT), :]
            tt = jax.lax.dot_general(xlb, xrb, (((1,), (1,)), ((), ())),
                                     preferred_element_type=jnp.float32)
            for q in range(k * gpt, min((k + 1) * gpt, nrows)):
                _gather(q)
            rr = 1.0 / tt
            v = rr[:, :128] + rr[:, 128:]
            while v.shape[0] > 8:
                h = v.shape[0] // 2
                v = v[:h, :] + v[h:, :]
            if k == 0 or k == _NK - 1:
                acc1 = acc1 + v
            else:
                acc2 = acc2 + v
        part = acc1 + 2.0 * acc2

        @pl.when(gid == 0)
        def _():
            acc_ref[...] = part

        @pl.when(gid > 0)
        def _():
            acc_ref[...] = acc_ref[...] + part

        zi = _softmax(ti_ref[...] + _gumbel(ni_ref[...]))
        zj = _softmax(tj_ref[...] + _gumbel(nj_ref[...]))
        dxt = jax.lax.dot_general(tw_ref[...], zi - zj,
                                  (((1,), (1,)), ((), ())),
                                  preferred_element_type=jnp.float32)
        den = jnp.float32(_ND) + jnp.sum(dxt * dxt, axis=0, keepdims=True)
        dof = pl.multiple_of(gid * _BB2, _BB2)
        den_ref[0:1, pl.ds(dof, _BB2)] = den

    @pl.when(gid == _G2)
    def _finish():
        ps = jnp.sum(acc_ref[...], axis=0, keepdims=True)
        tot = jnp.sum(ps, axis=1, keepdims=True)
        logpart = jnp.log(tot - jnp.float32(_NOBS))
        pij = pij_ref[...]
        out_ref[...] = pij * (jnp.log(pij) + jnp.log(den_ref[...]) + logpart)


def kernel(pij, noise_full, noise_i, noise_j, logits_weight, topic_w, topic_b, i, j):
    f32 = jnp.float32

    xl, xr = pl.pallas_call(
        _prep_body,
        grid=(_G1,),
        in_specs=[
            pl.BlockSpec((_BM1, _NT), lambda g: (g, 0)),
            pl.BlockSpec((_BM1, _NT), lambda g: (g, 0)),
            pl.BlockSpec((_ND, _NT), lambda g: (0, 0)),
            pl.BlockSpec((_ND, 1), lambda g: (0, 0)),
        ],
        out_specs=[
            pl.BlockSpec((_BM1, _NT), lambda g: (g, 0)),
            pl.BlockSpec((_BM1, _NT), lambda g: (g, 0)),
        ],
        out_shape=[
            jax.ShapeDtypeStruct((_NOBS, _NT), jnp.bfloat16),
            jax.ShapeDtypeStruct((_NOBS, _NT), jnp.bfloat16),
        ],
        compiler_params=pltpu.CompilerParams(
            dimension_semantics=("parallel",)),
        name="topic_prep",
    )(logits_weight, noise_full, topic_w, topic_b.reshape(_ND, 1))

    gclamp = _G2 - 1
    loss = pl.pallas_call(
        _main_body,
        grid=(_G2 + 1,),
        in_specs=[
            pl.BlockSpec((_BM2, _NT), lambda g: (jnp.minimum(g, gclamp), 0)),
            pl.BlockSpec((_NOBS, _NT), lambda g: (0, 0)),
            pl.BlockSpec((_NOBS, _NT), lambda g: (0, 0)),
            pl.BlockSpec((_BB2, _NT), lambda g: (jnp.minimum(g, gclamp), 0)),
            pl.BlockSpec((_BB2, _NT), lambda g: (jnp.minimum(g, gclamp), 0)),
            pl.BlockSpec(memory_space=pltpu.SMEM),
            pl.BlockSpec(memory_space=pltpu.SMEM),
            pl.BlockSpec((_ND, _NT), lambda g: (0, 0)),
            pl.BlockSpec((1, _B), lambda g: (0, 0)),
        ],
        out_specs=pl.BlockSpec((1, _B), lambda g: (0, 0)),
        out_shape=jax.ShapeDtypeStruct((1, _B), f32),
        scratch_shapes=[
            pltpu.VMEM((8, 128), f32),
            pltpu.VMEM((1, _B), f32),
            pltpu.VMEM((_BB2, _NT), f32),
            pltpu.VMEM((_BB2, _NT), f32),
        ],
        compiler_params=pltpu.CompilerParams(
            dimension_semantics=("arbitrary",)),
        name="topic_main",
    )(xl, xr, logits_weight, noise_i, noise_j,
      i.astype(jnp.int32), j.astype(jnp.int32), topic_w, pij.reshape(1, _B))

    return loss.reshape(_B)


# speedup vs baseline: 2.0154x; 1.0229x over previous
import jax
import jax.numpy as jnp
from jax.experimental import pallas as pl
from jax.experimental.pallas import tpu as pltpu

_EPS = 1e-9
_NOBS = 8192
_NT = 128
_ND = 64
_B = 8192

_G1 = 8
_BM1 = _NOBS // _G1

_G2 = 32
_BM2 = _NOBS // _G2
_BB2 = _B // _G2
_JT = 256
_NK = _G2 // 2 + 1


def _gumbel(u):
    return -jnp.log(-jnp.log(u + _EPS) + _EPS)


def _softmax(l):
    m = jnp.max(l, axis=-1, keepdims=True)
    e = jnp.exp(l - m)
    return e / jnp.sum(e, axis=-1, keepdims=True)


def _prep_body(logits_ref, noise_ref, w_ref, b_ref, xl_ref, xr_ref):
    z = _softmax(logits_ref[...] + _gumbel(noise_ref[...]))
    wb = w_ref[...] + b_ref[...]
    wbp = jnp.concatenate([wb, jnp.zeros((_NT - _ND, _NT), jnp.float32)],
                          axis=0)
    x = jax.lax.dot_general(z, wbp, (((1,), (1,)), ((), ())),
                            preferred_element_type=jnp.float32)
    sq = jnp.sum(x * x, axis=-1, keepdims=True)
    lane = jax.lax.broadcasted_iota(jnp.int32, (_BM1, _NT), 1)
    e64 = lane == _ND
    e65 = lane == _ND + 1
    aug_l = jnp.where(e64, 1.0 + sq, jnp.where(e65, 1.0, 0.0))
    aug_r = jnp.where(e64, 1.0, jnp.where(e65, sq, 0.0))
    xl_ref[...] = (aug_l - 2.0 * x).astype(jnp.bfloat16)
    xr_ref[...] = (x + aug_r).astype(jnp.bfloat16)


def _main_body(xl_ref, xr_ref, lw_ref, ni_ref, nj_ref,
               sti_ref, ami_ref, stj_ref, amj_ref,
               tw_ref, pij_ref, out_ref, acc_ref, den_ref, ti_ref, tj_ref):
    gid = pl.program_id(0)

    @pl.when(gid < _G2)
    def _work():
        gbase = gid * _BB2
        nrows = 2 * _BB2
        ngt = 11
        gpt = -(-nrows // ngt)

        def _gather(q):
            r = q // 2
            st_ref, am_ref, dst = ((sti_ref, ami_ref, ti_ref) if q % 2 == 0
                                   else (stj_ref, amj_ref, tj_ref))
            st = pl.multiple_of(st_ref[gbase + r], 8)
            chunk = lw_ref[pl.ds(st, 8), :]
            rolled = pltpu.roll(chunk, am_ref[gbase + r], axis=0)
            dst[r:r + 1, :] = rolled[0:1, :]

        def _den_chunk(c):
            sl = slice(c * 128, (c + 1) * 128)
            zi = _softmax(ti_ref[sl, :] + _gumbel(ni_ref[sl, :]))
            zj = _softmax(tj_ref[sl, :] + _gumbel(nj_ref[sl, :]))
            dxt = jax.lax.dot_general(tw_ref[...], zi - zj,
                                      (((1,), (1,)), ((), ())),
                                      preferred_element_type=jnp.float32)
            dc = jnp.float32(_ND) + jnp.sum(dxt * dxt, axis=0, keepdims=True)
            dof = pl.multiple_of(gid * _BB2 + c * 128, 128)
            den_ref[0:1, pl.ds(dof, 128)] = dc

        xlb = xl_ref[...]
        acc1 = jnp.zeros((8, 128), jnp.float32)
        acc2 = jnp.zeros((8, 128), jnp.float32)
        for k in range(_NK):
            jo = pl.multiple_of(((gid + k) & (_G2 - 1)) * _JT, _JT)
            xrb = xr_ref[pl.ds(jo, _JT), :]
            tt = jax.lax.dot_general(xlb, xrb, (((1,), (1,)), ((), ())),
                                     preferred_element_type=jnp.float32)
            for q in range(k * gpt, min((k + 1) * gpt, nrows)):
                _gather(q)
            if k == 6:
                _den_chunk(0)
            if k == 11:
                _den_chunk(1)
            rr = 1.0 / tt
            v = rr[:, :128] + rr[:, 128:]
            while v.shape[0] > 8:
                h = v.shape[0] // 2
                v = v[:h, :] + v[h:, :]
            if k == 0 or k == _NK - 1:
                acc1 = acc1 + v
            else:
                acc2 = acc2 + v
        part = acc1 + 2.0 * acc2

        @pl.when(gid == 0)
        def _():
            acc_ref[...] = part

        @pl.when(gid > 0)
        def _():
            acc_ref[...] = acc_ref[...] + part

    @pl.when(gid == _G2)
    def _finish():
        ps = jnp.sum(acc_ref[...], axis=0, keepdims=True)
        tot = jnp.sum(ps, axis=1, keepdims=True)
        logpart = jnp.log(tot - jnp.float32(_NOBS))
        pij = pij_ref[...]
        out_ref[...] = pij * (jnp.log(pij) + jnp.log(den_ref[...]) + logpart)


def kernel(pij, noise_full, noise_i, noise_j, logits_weight, topic_w, topic_b, i, j):
    f32 = jnp.float32
    i32 = i.astype(jnp.int32)
    j32 = j.astype(jnp.int32)
    sti = (i32 >> 3) << 3
    ami = (-i32) & 7
    stj = (j32 >> 3) << 3
    amj = (-j32) & 7

    xl, xr = pl.pallas_call(
        _prep_body,
        grid=(_G1,),
        in_specs=[
            pl.BlockSpec((_BM1, _NT), lambda g: (g, 0)),
            pl.BlockSpec((_BM1, _NT), lambda g: (g, 0)),
            pl.BlockSpec((_ND, _NT), lambda g: (0, 0)),
            pl.BlockSpec((_ND, 1), lambda g: (0, 0)),
        ],
        out_specs=[
            pl.BlockSpec((_BM1, _NT), lambda g: (g, 0)),
            pl.BlockSpec((_BM1, _NT), lambda g: (g, 0)),
        ],
        out_shape=[
            jax.ShapeDtypeStruct((_NOBS, _NT), jnp.bfloat16),
            jax.ShapeDtypeStruct((_NOBS, _NT), jnp.bfloat16),
        ],
        compiler_params=pltpu.CompilerParams(
            dimension_semantics=("parallel",)),
        name="topic_prep",
    )(logits_weight, noise_full, topic_w, topic_b.reshape(_ND, 1))

    gclamp = _G2 - 1
    loss = pl.pallas_call(
        _main_body,
        grid=(_G2 + 1,),
        in_specs=[
            pl.BlockSpec((_BM2, _NT), lambda g: (jnp.minimum(g, gclamp), 0)),
            pl.BlockSpec((_NOBS, _NT), lambda g: (0, 0)),
            pl.BlockSpec((_NOBS, _NT), lambda g: (0, 0)),
            pl.BlockSpec((_BB2, _NT), lambda g: (jnp.minimum(g, gclamp), 0)),
            pl.BlockSpec((_BB2, _NT), lambda g: (jnp.minimum(g, gclamp), 0)),
            pl.BlockSpec(memory_space=pltpu.SMEM),
            pl.BlockSpec(memory_space=pltpu.SMEM),
            pl.BlockSpec(memory_space=pltpu.SMEM),
            pl.BlockSpec(memory_space=pltpu.SMEM),
            pl.BlockSpec((_ND, _NT), lambda g: (0, 0)),
            pl.BlockSpec((1, _B), lambda g: (0, 0)),
        ],
        out_specs=pl.BlockSpec((1, _B), lambda g: (0, 0)),
        out_shape=jax.ShapeDtypeStruct((1, _B), f32),
        scratch_shapes=[
            pltpu.VMEM((8, 128), f32),
            pltpu.VMEM((1, _B), f32),
            pltpu.VMEM((_BB2, _NT), f32),
            pltpu.VMEM((_BB2, _NT), f32),
        ],
        compiler_params=pltpu.CompilerParams(
            dimension_semantics=("arbitrary",)),
        name="topic_main",
    )(xl, xr, logits_weight, noise_i, noise_j,
      sti, ami, stj, amj, topic_w, pij.reshape(1, _B))

    return loss.reshape(_B)


# speedup vs baseline: 2.2353x; 1.1091x over previous
import jax
import jax.numpy as jnp
from jax.experimental import pallas as pl
from jax.experimental.pallas import tpu as pltpu

_EPS = 1e-9
_NOBS = 8192
_NT = 128
_ND = 64
_B = 8192

_G1 = 8
_BM1 = _NOBS // _G1

_G2 = 32
_BM2 = _NOBS // _G2
_BB2 = _B // _G2
_JT = 256
_NK = _G2 // 2 + 1


def _gumbel(u):
    return -jnp.log(-jnp.log(u + _EPS) + _EPS)


def _softmax(l):
    m = jnp.max(l, axis=-1, keepdims=True)
    e = jnp.exp(l - m)
    return e / jnp.sum(e, axis=-1, keepdims=True)


def _prep_body(logits_ref, noise_ref, w_ref, b_ref, xl_ref, xr_ref):
    z = _softmax(logits_ref[...] + _gumbel(noise_ref[...]))
    wb = w_ref[...] + b_ref[...]
    wbp = jnp.concatenate([wb, jnp.zeros((_NT - _ND, _NT), jnp.float32)],
                          axis=0)
    x = jax.lax.dot_general(z, wbp, (((1,), (1,)), ((), ())),
                            preferred_element_type=jnp.float32)
    sq = jnp.sum(x * x, axis=-1, keepdims=True)
    lane = jax.lax.broadcasted_iota(jnp.int32, (_BM1, _NT), 1)
    e64 = lane == _ND
    e65 = lane == _ND + 1
    aug_l = jnp.where(e64, 1.0 + sq, jnp.where(e65, 1.0, 0.0))
    aug_r = jnp.where(e64, 1.0, jnp.where(e65, sq, 0.0))
    xl_ref[...] = (aug_l - 2.0 * x).astype(jnp.bfloat16)
    xr_ref[...] = (x + aug_r).astype(jnp.bfloat16)


def _main_body(xl_ref, xr_ref, lw_ref, ni_ref, nj_ref,
               sti_ref, ami_ref, stj_ref, amj_ref,
               tw_ref, pij_ref, out_ref, acc_ref, den_ref, ti_ref, tj_ref):
    gid = pl.program_id(0)
    nsteps = _G2 // 2

    @pl.when(gid < nsteps)
    def _work():
        rows = 2 * _BB2
        gbase = gid * rows
        nq = 2 * rows
        ngt = 22
        gpt = -(-nq // ngt)

        def _gather(q):
            r = q // 2
            st_ref, am_ref, dst = ((sti_ref, ami_ref, ti_ref) if q % 2 == 0
                                   else (stj_ref, amj_ref, tj_ref))
            st = pl.multiple_of(st_ref[gbase + r], 8)
            chunk = lw_ref[pl.ds(st, 8), :]
            rolled = pltpu.roll(chunk, am_ref[gbase + r], axis=0)
            dst[r:r + 1, :] = rolled[0:1, :]

        def _den_chunk(c):
            sl = slice(c * 128, (c + 1) * 128)
            zi = _softmax(ti_ref[sl, :] + _gumbel(ni_ref[sl, :]))
            zj = _softmax(tj_ref[sl, :] + _gumbel(nj_ref[sl, :]))
            dxt = jax.lax.dot_general(tw_ref[...], zi - zj,
                                      (((1,), (1,)), ((), ())),
                                      preferred_element_type=jnp.float32)
            dc = jnp.float32(_ND) + jnp.sum(dxt * dxt, axis=0, keepdims=True)
            dof = pl.multiple_of(gbase + c * 128, 128)
            den_ref[0:1, pl.ds(dof, 128)] = dc

        acc1 = jnp.zeros((8, 128), jnp.float32)
        acc2 = jnp.zeros((8, 128), jnp.float32)
        den_at = (6, 11, 17, 22)
        for t in range(2 * _NK):
            s, k = divmod(t, _NK)
            blk = 2 * gid + s
            xlb = xl_ref[s * _BM2:(s + 1) * _BM2, :]
            jo = pl.multiple_of(((blk + k) & (_G2 - 1)) * _JT, _JT)
            xrb = xr_ref[pl.ds(jo, _JT), :]
            tt = jax.lax.dot_general(xlb, xrb, (((1,), (1,)), ((), ())),
                                     preferred_element_type=jnp.float32)
            for q in range(t * gpt, min((t + 1) * gpt, nq)):
                _gather(q)
            if t in den_at:
                _den_chunk(den_at.index(t))
            rr = 1.0 / tt
            v = rr[:, :128] + rr[:, 128:]
            while v.shape[0] > 8:
                h = v.shape[0] // 2
                v = v[:h, :] + v[h:, :]
            if k == 0 or k == _NK - 1:
                acc1 = acc1 + v
            else:
                acc2 = acc2 + v
        part = acc1 + 2.0 * acc2

        @pl.when(gid == 0)
        def _():
            acc_ref[...] = part

        @pl.when(gid > 0)
        def _():
            acc_ref[...] = acc_ref[...] + part

    @pl.when(gid == nsteps)
    def _finish():
        ps = jnp.sum(acc_ref[...], axis=0, keepdims=True)
        tot = jnp.sum(ps, axis=1, keepdims=True)
        logpart = jnp.log(tot - jnp.float32(_NOBS))
        pij = pij_ref[...]
        out_ref[...] = pij * (jnp.log(pij) + jnp.log(den_ref[...]) + logpart)


def kernel(pij, noise_full, noise_i, noise_j, logits_weight, topic_w, topic_b, i, j):
    f32 = jnp.float32
    i32 = i.astype(jnp.int32)
    j32 = j.astype(jnp.int32)
    sti = (i32 >> 3) << 3
    ami = (-i32) & 7
    stj = (j32 >> 3) << 3
    amj = (-j32) & 7

    xl, xr = pl.pallas_call(
        _prep_body,
        grid=(_G1,),
        in_specs=[
            pl.BlockSpec((_BM1, _NT), lambda g: (g, 0)),
            pl.BlockSpec((_BM1, _NT), lambda g: (g, 0)),
            pl.BlockSpec((_ND, _NT), lambda g: (0, 0)),
            pl.BlockSpec((_ND, 1), lambda g: (0, 0)),
        ],
        out_specs=[
            pl.BlockSpec((_BM1, _NT), lambda g: (g, 0)),
            pl.BlockSpec((_BM1, _NT), lambda g: (g, 0)),
        ],
        out_shape=[
            jax.ShapeDtypeStruct((_NOBS, _NT), jnp.bfloat16),
            jax.ShapeDtypeStruct((_NOBS, _NT), jnp.bfloat16),
        ],
        compiler_params=pltpu.CompilerParams(
            dimension_semantics=("parallel",)),
        name="topic_prep",
    )(logits_weight, noise_full, topic_w, topic_b.reshape(_ND, 1))

    gclamp = _G2 // 2 - 1
    loss = pl.pallas_call(
        _main_body,
        grid=(_G2 // 2 + 1,),
        in_specs=[
            pl.BlockSpec((2 * _BM2, _NT),
                         lambda g: (jnp.minimum(g, gclamp), 0)),
            pl.BlockSpec((_NOBS, _NT), lambda g: (0, 0)),
            pl.BlockSpec((_NOBS, _NT), lambda g: (0, 0)),
            pl.BlockSpec((2 * _BB2, _NT),
                         lambda g: (jnp.minimum(g, gclamp), 0)),
            pl.BlockSpec((2 * _BB2, _NT),
                         lambda g: (jnp.minimum(g, gclamp), 0)),
            pl.BlockSpec(memory_space=pltpu.SMEM),
            pl.BlockSpec(memory_space=pltpu.SMEM),
            pl.BlockSpec(memory_space=pltpu.SMEM),
            pl.BlockSpec(memory_space=pltpu.SMEM),
            pl.BlockSpec((_ND, _NT), lambda g: (0, 0)),
            pl.BlockSpec((1, _B), lambda g: (0, 0)),
        ],
        out_specs=pl.BlockSpec((1, _B), lambda g: (0, 0)),
        out_shape=jax.ShapeDtypeStruct((1, _B), f32),
        scratch_shapes=[
            pltpu.VMEM((8, 128), f32),
            pltpu.VMEM((1, _B), f32),
            pltpu.VMEM((2 * _BB2, _NT), f32),
            pltpu.VMEM((2 * _BB2, _NT), f32),
        ],
        compiler_params=pltpu.CompilerParams(
            dimension_semantics=("arbitrary",)),
        name="topic_main",
    )(xl, xr, logits_weight, noise_i, noise_j,
      sti, ami, stj, amj, topic_w, pij.reshape(1, _B))

    return loss.reshape(_B)


# speedup vs baseline: 2.2513x; 1.0072x over previous
import jax
import jax.numpy as jnp
from jax.experimental import pallas as pl
from jax.experimental.pallas import tpu as pltpu

_EPS = 1e-9
_NOBS = 8192
_NT = 128
_ND = 64
_B = 8192

_G1 = 8
_BM1 = _NOBS // _G1

_G2 = 32
_BM2 = _NOBS // _G2
_BB2 = _B // _G2
_JT = 256
_NK = _G2 // 2 + 1
_RB = 4


def _gumbel(u):
    return -jnp.log(-jnp.log(u + _EPS) + _EPS)


def _softmax(l):
    m = jnp.max(l, axis=-1, keepdims=True)
    e = jnp.exp(l - m)
    return e / jnp.sum(e, axis=-1, keepdims=True)


def _prep_body(logits_ref, noise_ref, w_ref, b_ref, xl_ref, xr_ref):
    z = _softmax(logits_ref[...] + _gumbel(noise_ref[...]))
    wb = w_ref[...] + b_ref[...]
    wbp = jnp.concatenate([wb, jnp.zeros((_NT - _ND, _NT), jnp.float32)],
                          axis=0)
    x = jax.lax.dot_general(z, wbp, (((1,), (1,)), ((), ())),
                            preferred_element_type=jnp.float32)
    sq = jnp.sum(x * x, axis=-1, keepdims=True)
    lane = jax.lax.broadcasted_iota(jnp.int32, (_BM1, _NT), 1)
    e64 = lane == _ND
    e65 = lane == _ND + 1
    aug_l = jnp.where(e64, 1.0 + sq, jnp.where(e65, 1.0, 0.0))
    aug_r = jnp.where(e64, 1.0, jnp.where(e65, sq, 0.0))
    xl_ref[...] = (aug_l - 2.0 * x).astype(jnp.bfloat16)
    xr_ref[...] = (x + aug_r).astype(jnp.bfloat16)


def _main_body(xl_ref, xr_ref, lw_ref, ni_ref, nj_ref,
               sti_ref, ami_ref, stj_ref, amj_ref,
               tw_ref, pij_ref, out_ref, acc_ref, den_ref, ti_ref, tj_ref):
    gid = pl.program_id(0)
    nsteps = _G2 // _RB

    @pl.when(gid < nsteps)
    def _work():
        rows = _RB * _BB2
        gbase = gid * rows
        nq = 2 * rows
        ntiles = _RB * _NK
        ngt = (2 * ntiles) // 3
        gpt = -(-nq // ngt)
        den_at = {-(-(256 * (c + 1)) // gpt): c for c in range(rows // 128)}

        def _gather(q):
            r = q // 2
            st_ref, am_ref, dst = ((sti_ref, ami_ref, ti_ref) if q % 2 == 0
                                   else (stj_ref, amj_ref, tj_ref))
            st = pl.multiple_of(st_ref[gbase + r], 8)
            chunk = lw_ref[pl.ds(st, 8), :]
            rolled = pltpu.roll(chunk, am_ref[gbase + r], axis=0)
            dst[r:r + 1, :] = rolled[0:1, :]

        def _den_chunk(c):
            sl = slice(c * 128, (c + 1) * 128)
            zi = _softmax(ti_ref[sl, :] + _gumbel(ni_ref[sl, :]))
            zj = _softmax(tj_ref[sl, :] + _gumbel(nj_ref[sl, :]))
            dxt = jax.lax.dot_general(tw_ref[...], zi - zj,
                                      (((1,), (1,)), ((), ())),
                                      preferred_element_type=jnp.float32)
            dc = jnp.float32(_ND) + jnp.sum(dxt * dxt, axis=0, keepdims=True)
            dof = pl.multiple_of(gbase + c * 128, 128)
            den_ref[0:1, pl.ds(dof, 128)] = dc

        acc1 = jnp.zeros((8, 128), jnp.float32)
        acc2 = jnp.zeros((8, 128), jnp.float32)
        for t in range(ntiles):
            s, k = divmod(t, _NK)
            blk = _RB * gid + s
            xlb = xl_ref[s * _BM2:(s + 1) * _BM2, :]
            jo = pl.multiple_of(((blk + k) & (_G2 - 1)) * _JT, _JT)
            xrb = xr_ref[pl.ds(jo, _JT), :]
            tt = jax.lax.dot_general(xlb, xrb, (((1,), (1,)), ((), ())),
                                     preferred_element_type=jnp.float32)
            for q in range(t * gpt, min((t + 1) * gpt, nq)):
                _gather(q)
            if t in den_at:
                _den_chunk(den_at[t])
            rr = 1.0 / tt
            v = rr[:, :128] + rr[:, 128:]
            while v.shape[0] > 8:
                h = v.shape[0] // 2
                v = v[:h, :] + v[h:, :]
            if k == 0 or k == _NK - 1:
                acc1 = acc1 + v
            else:
                acc2 = acc2 + v
        part = acc1 + 2.0 * acc2

        @pl.when(gid == 0)
        def _():
            acc_ref[...] = part

        @pl.when(gid > 0)
        def _():
            acc_ref[...] = acc_ref[...] + part

    @pl.when(gid == nsteps)
    def _finish():
        ps = jnp.sum(acc_ref[...], axis=0, keepdims=True)
        tot = jnp.sum(ps, axis=1, keepdims=True)
        logpart = jnp.log(tot - jnp.float32(_NOBS))
        pij = pij_ref[...]
        out_ref[...] = pij * (jnp.log(pij) + jnp.log(den_ref[...]) + logpart)


def kernel(pij, noise_full, noise_i, noise_j, logits_weight, topic_w, topic_b, i, j):
    f32 = jnp.float32
    i32 = i.astype(jnp.int32)
    j32 = j.astype(jnp.int32)
    sti = (i32 >> 3) << 3
    ami = (-i32) & 7
    stj = (j32 >> 3) << 3
    amj = (-j32) & 7

    xl, xr = pl.pallas_call(
        _prep_body,
        grid=(_G1,),
        in_specs=[
            pl.BlockSpec((_BM1, _NT), lambda g: (g, 0)),
            pl.BlockSpec((_BM1, _NT), lambda g: (g, 0)),
            pl.BlockSpec((_ND, _NT), lambda g: (0, 0)),
            pl.BlockSpec((_ND, 1), lambda g: (0, 0)),
        ],
        out_specs=[
            pl.BlockSpec((_BM1, _NT), lambda g: (g, 0)),
            pl.BlockSpec((_BM1, _NT), lambda g: (g, 0)),
        ],
        out_shape=[
            jax.ShapeDtypeStruct((_NOBS, _NT), jnp.bfloat16),
            jax.ShapeDtypeStruct((_NOBS, _NT), jnp.bfloat16),
        ],
        compiler_params=pltpu.CompilerParams(
            dimension_semantics=("parallel",)),
        name="topic_prep",
    )(logits_weight, noise_full, topic_w, topic_b.reshape(_ND, 1))

    gclamp = _G2 // _RB - 1
    loss = pl.pallas_call(
        _main_body,
        grid=(_G2 // _RB + 1,),
        in_specs=[
            pl.BlockSpec((_RB * _BM2, _NT),
                         lambda g: (jnp.minimum(g, gclamp), 0)),
            pl.BlockSpec((_NOBS, _NT), lambda g: (0, 0)),
            pl.BlockSpec((_NOBS, _NT), lambda g: (0, 0)),
            pl.BlockSpec((_RB * _BB2, _NT),
                         lambda g: (jnp.minimum(g, gclamp), 0)),
            pl.BlockSpec((_RB * _BB2, _NT),
                         lambda g: (jnp.minimum(g, gclamp), 0)),
            pl.BlockSpec(memory_space=pltpu.SMEM),
            pl.BlockSpec(memory_space=pltpu.SMEM),
            pl.BlockSpec(memory_space=pltpu.SMEM),
            pl.BlockSpec(memory_space=pltpu.SMEM),
            pl.BlockSpec((_ND, _NT), lambda g: (0, 0)),
            pl.BlockSpec((1, _B), lambda g: (0, 0)),
        ],
        out_specs=pl.BlockSpec((1, _B), lambda g: (0, 0)),
        out_shape=jax.ShapeDtypeStruct((1, _B), f32),
        scratch_shapes=[
            pltpu.VMEM((8, 128), f32),
            pltpu.VMEM((1, _B), f32),
            pltpu.VMEM((_RB * _BB2, _NT), f32),
            pltpu.VMEM((_RB * _BB2, _NT), f32),
        ],
        compiler_params=pltpu.CompilerParams(
            dimension_semantics=("arbitrary",)),
        name="topic_main",
    )(xl, xr, logits_weight, noise_i, noise_j,
      sti, ami, stj, amj, topic_w, pij.reshape(1, _B))

    return loss.reshape(_B)


# speedup vs baseline: 2.3525x; 1.0449x over previous
import jax
import jax.numpy as jnp
from jax.experimental import pallas as pl
from jax.experimental.pallas import tpu as pltpu

_EPS = 1e-9
_NOBS = 8192
_NT = 128
_ND = 64
_B = 8192

_G1 = 8
_BM1 = _NOBS // _G1

_G2 = 32
_BM2 = _NOBS // _G2
_BB2 = _B // _G2
_JT = 256
_NK = _G2 // 2 + 1
_RB = 4


def _gumbel(u):
    return -jnp.log(-jnp.log(u + _EPS) + _EPS)


def _softmax(l):
    m = jnp.max(l, axis=-1, keepdims=True)
    e = jnp.exp(l - m)
    return e / jnp.sum(e, axis=-1, keepdims=True)


def _prep_block(logits_ref, noise_ref, w_ref, b_ref, xl_ref, xr_ref, g):
    sl = slice(0, _BM1)
    z = _softmax(logits_ref[...] + _gumbel(noise_ref[...]))
    wb = w_ref[...] + b_ref[...]
    wbp = jnp.concatenate([wb, jnp.zeros((_NT - _ND, _NT), jnp.float32)],
                          axis=0)
    x = jax.lax.dot_general(z, wbp, (((1,), (1,)), ((), ())),
                            preferred_element_type=jnp.float32)
    sq = jnp.sum(x * x, axis=-1, keepdims=True)
    lane = jax.lax.broadcasted_iota(jnp.int32, (_BM1, _NT), 1)
    e64 = lane == _ND
    e65 = lane == _ND + 1
    aug_l = jnp.where(e64, 1.0 + sq, jnp.where(e65, 1.0, 0.0))
    aug_r = jnp.where(e64, 1.0, jnp.where(e65, sq, 0.0))
    ro = pl.multiple_of(g * _BM1, _BM1)
    xl_ref[pl.ds(ro, _BM1), :] = (aug_l - 2.0 * x).astype(jnp.bfloat16)
    xr_ref[pl.ds(ro, _BM1), :] = (x + aug_r).astype(jnp.bfloat16)


def _main_body(lg_ref, nf_ref, w_ref, b_ref, lw_ref, ni_ref, nj_ref,
               sti_ref, ami_ref, stj_ref, amj_ref,
               tw_ref, pij_ref, out_ref,
               acc_ref, den_ref, ti_ref, tj_ref, xl_ref, xr_ref):
    gid = pl.program_id(0)
    nsteps = _G2 // _RB

    @pl.when(gid < _G1)
    def _prep():
        _prep_block(lg_ref, nf_ref, w_ref, b_ref, xl_ref, xr_ref, gid)

    @pl.when(jnp.logical_and(gid >= _G1, gid < _G1 + nsteps))
    def _work():
        gid2 = gid - _G1
        rows = _RB * _BB2
        gbase = gid2 * rows
        nq = 2 * rows
        ntiles = _RB * _NK
        ngt = (2 * ntiles) // 3
        gpt = -(-nq // ngt)
        den_at = {-(-(256 * (c + 1)) // gpt): c for c in range(rows // 128)}

        def _gather(q):
            r = q // 2
            st_ref, am_ref, dst = ((sti_ref, ami_ref, ti_ref) if q % 2 == 0
                                   else (stj_ref, amj_ref, tj_ref))
            st = pl.multiple_of(st_ref[gbase + r], 8)
            chunk = lw_ref[pl.ds(st, 8), :]
            rolled = pltpu.roll(chunk, am_ref[gbase + r], axis=0)
            dst[r:r + 1, :] = rolled[0:1, :]

        def _den_chunk(c):
            sl = slice(c * 128, (c + 1) * 128)
            zi = _softmax(ti_ref[sl, :] + _gumbel(ni_ref[sl, :]))
            zj = _softmax(tj_ref[sl, :] + _gumbel(nj_ref[sl, :]))
            dxt = jax.lax.dot_general(tw_ref[...], zi - zj,
                                      (((1,), (1,)), ((), ())),
                                      preferred_element_type=jnp.float32)
            dc = jnp.float32(_ND) + jnp.sum(dxt * dxt, axis=0, keepdims=True)
            dof = pl.multiple_of(gbase + c * 128, 128)
            den_ref[0:1, pl.ds(dof, 128)] = dc

        acc1 = jnp.zeros((8, 128), jnp.float32)
        acc2 = jnp.zeros((8, 128), jnp.float32)
        for t in range(ntiles):
            s, k = divmod(t, _NK)
            blk = _RB * gid2 + s
            ro = pl.multiple_of(blk * _BM2, _BM2)
            xlb = xl_ref[pl.ds(ro, _BM2), :]
            jo = pl.multiple_of(((blk + k) & (_G2 - 1)) * _JT, _JT)
            xrb = xr_ref[pl.ds(jo, _JT), :]
            tt = jax.lax.dot_general(xlb, xrb, (((1,), (1,)), ((), ())),
                                     preferred_element_type=jnp.float32)
            for q in range(t * gpt, min((t + 1) * gpt, nq)):
                _gather(q)
            if t in den_at:
                _den_chunk(den_at[t])
            rr = 1.0 / tt
            v = rr[:, :128] + rr[:, 128:]
            while v.shape[0] > 8:
                h = v.shape[0] // 2
                v = v[:h, :] + v[h:, :]
            if k == 0 or k == _NK - 1:
                acc1 = acc1 + v
            else:
                acc2 = acc2 + v
        part = acc1 + 2.0 * acc2

        @pl.when(gid == _G1)
        def _():
            acc_ref[...] = part

        @pl.when(gid > _G1)
        def _():
            acc_ref[...] = acc_ref[...] + part

    @pl.when(gid == _G1 + nsteps)
    def _finish():
        ps = jnp.sum(acc_ref[...], axis=0, keepdims=True)
        tot = jnp.sum(ps, axis=1, keepdims=True)
        logpart = jnp.log(tot - jnp.float32(_NOBS))
        pij = pij_ref[...]
        out_ref[...] = pij * (jnp.log(pij) + jnp.log(den_ref[...]) + logpart)


def kernel(pij, noise_full, noise_i, noise_j, logits_weight, topic_w, topic_b, i, j):
    f32 = jnp.float32
    i32 = i.astype(jnp.int32)
    j32 = j.astype(jnp.int32)
    sti = (i32 >> 3) << 3
    ami = (-i32) & 7
    stj = (j32 >> 3) << 3
    amj = (-j32) & 7

    nsteps = _G2 // _RB
    gclamp = nsteps - 1

    def _pclamp(g):
        return (jnp.minimum(g, _G1 - 1), 0)

    def _mclamp(g):
        return (jnp.clip(g - _G1, 0, gclamp), 0)

    loss = pl.pallas_call(
        _main_body,
        grid=(_G1 + nsteps + 1,),
        in_specs=[
            pl.BlockSpec((_BM1, _NT), _pclamp),
            pl.BlockSpec((_BM1, _NT), _pclamp),
            pl.BlockSpec((_ND, _NT), lambda g: (0, 0)),
            pl.BlockSpec((_ND, 1), lambda g: (0, 0)),
            pl.BlockSpec((_NOBS, _NT), lambda g: (0, 0)),
            pl.BlockSpec((_RB * _BB2, _NT), _mclamp),
            pl.BlockSpec((_RB * _BB2, _NT), _mclamp),
            pl.BlockSpec(memory_space=pltpu.SMEM),
            pl.BlockSpec(memory_space=pltpu.SMEM),
            pl.BlockSpec(memory_space=pltpu.SMEM),
            pl.BlockSpec(memory_space=pltpu.SMEM),
            pl.BlockSpec((_ND, _NT), lambda g: (0, 0)),
            pl.BlockSpec((1, _B), lambda g: (0, 0)),
        ],
        out_specs=pl.BlockSpec((1, _B), lambda g: (0, 0)),
        out_shape=jax.ShapeDtypeStruct((1, _B), f32),
        scratch_shapes=[
            pltpu.VMEM((8, 128), f32),
            pltpu.VMEM((1, _B), f32),
            pltpu.VMEM((_RB * _BB2, _NT), f32),
            pltpu.VMEM((_RB * _BB2, _NT), f32),
            pltpu.VMEM((_NOBS, _NT), jnp.bfloat16),
            pltpu.VMEM((_NOBS, _NT), jnp.bfloat16),
        ],
        compiler_params=pltpu.CompilerParams(
            dimension_semantics=("arbitrary",)),
        name="topic_fused",
    )(logits_weight, noise_full, topic_w, topic_b.reshape(_ND, 1),
      logits_weight, noise_i, noise_j,
      sti, ami, stj, amj, topic_w, pij.reshape(1, _B))

    return loss.reshape(_B)


# speedup vs baseline: 2.3619x; 1.0040x over previous
import jax
import jax.numpy as jnp
from jax.experimental import pallas as pl
from jax.experimental.pallas import tpu as pltpu

_EPS = 1e-9
_NOBS = 8192
_NT = 128
_ND = 64
_B = 8192

_G1 = 8
_BM1 = _NOBS // _G1

_G2 = 32
_BM2 = _NOBS // _G2
_BB2 = _B // _G2
_JT = 256
_NK = _G2 // 2 + 1
_RB = 4


def _gumbel(u):
    return -jnp.log(-jnp.log(u + _EPS) + _EPS)


def _softmax(l):
    m = jnp.max(l, axis=-1, keepdims=True)
    e = jnp.exp(l - m)
    return e / jnp.sum(e, axis=-1, keepdims=True)


def _prep_block(logits_ref, noise_ref, w_ref, b_ref, xl_ref, xr_ref, g):
    z = _softmax(logits_ref[...] + _gumbel(noise_ref[...]))
    wb = w_ref[...] + b_ref[...]
    wbp = jnp.concatenate([wb, jnp.zeros((_NT - _ND, _NT), jnp.float32)],
                          axis=0)
    x = jax.lax.dot_general(z, wbp, (((1,), (1,)), ((), ())),
                            preferred_element_type=jnp.float32)
    sq = jnp.sum(x * x, axis=-1, keepdims=True)
    lane = jax.lax.broadcasted_iota(jnp.int32, (_BM1, _NT), 1)
    e64 = lane == _ND
    e65 = lane == _ND + 1
    aug_l = jnp.where(e64, 1.0 + sq, jnp.where(e65, 1.0, 0.0))
    aug_r = jnp.where(e64, 1.0, jnp.where(e65, sq, 0.0))
    ro = pl.multiple_of(g * _BM1, _BM1)
    xl_ref[pl.ds(ro, _BM1), :] = (aug_l - 2.0 * x).astype(jnp.bfloat16)
    xr_ref[pl.ds(ro, _BM1), :] = (x + aug_r).astype(jnp.bfloat16)


def _main_body(lg_ref, nf_ref, w_ref, b_ref, lw_ref, ni_ref, nj_ref,
               sti_ref, ami_ref, stj_ref, amj_ref,
               tw_ref, pij_ref, out_ref,
               acc_ref, den_ref, ti_ref, tj_ref, xl_ref, xr_ref):
    gid = pl.program_id(0)
    nsteps = _G2 // _RB

    @pl.when(gid < _G1)
    def _prep():
        _prep_block(lg_ref, nf_ref, w_ref, b_ref, xl_ref, xr_ref, gid)

    @pl.when(jnp.logical_and(gid >= _G1, gid < _G1 + nsteps))
    def _work():
        gid2 = gid - _G1
        rows = _RB * _BB2
        gbase = gid2 * rows
        nq = 2 * rows
        ntiles = _RB * _NK
        ngt = (2 * ntiles) // 3
        gpt = -(-nq // ngt)
        den_at = {-(-(256 * (c + 1)) // gpt): c for c in range(rows // 128)}

        def _gather(q):
            r = q // 2
            st_ref, am_ref, dst = ((sti_ref, ami_ref, ti_ref) if q % 2 == 0
                                   else (stj_ref, amj_ref, tj_ref))
            st = pl.multiple_of(st_ref[gbase + r], 8)
            chunk = lw_ref[pl.ds(st, 8), :]
            rolled = pltpu.roll(chunk, am_ref[gbase + r], axis=0)
            dst[r:r + 1, :] = rolled[0:1, :]

        def _den_chunk(c):
            sl = slice(c * 128, (c + 1) * 128)
            zi = _softmax(ti_ref[sl, :] + _gumbel(ni_ref[sl, :]))
            zj = _softmax(tj_ref[sl, :] + _gumbel(nj_ref[sl, :]))
            dxt = jax.lax.dot_general(tw_ref[...], zi - zj,
                                      (((1,), (1,)), ((), ())),
                                      preferred_element_type=jnp.float32)
            dc = jnp.float32(_ND) + jnp.sum(dxt * dxt, axis=0, keepdims=True)
            dof = pl.multiple_of(gbase + c * 128, 128)
            den_ref[0:1, pl.ds(dof, 128)] = dc

        acc1 = jnp.zeros((8, 128), jnp.float32)
        acc2 = jnp.zeros((8, 128), jnp.float32)
        for t in range(ntiles):
            s, k = divmod(t, _NK)
            blk = _RB * gid2 + s
            ro = pl.multiple_of(blk * _BM2, _BM2)
            xlb = xl_ref[pl.ds(ro, _BM2), :]
            jo = pl.multiple_of(((blk + k) & (_G2 - 1)) * _JT, _JT)
            xrb = xr_ref[pl.ds(jo, _JT), :]
            tt = jax.lax.dot_general(xlb, xrb, (((1,), (1,)), ((), ())),
                                     preferred_element_type=jnp.float32)
            for q in range(t * gpt, min((t + 1) * gpt, nq)):
                _gather(q)
            if t in den_at:
                _den_chunk(den_at[t])
            rr = 1.0 / tt
            v = rr[:, :128] + rr[:, 128:]
            while v.shape[0] > 8:
                h = v.shape[0] // 2
                v = v[:h, :] + v[h:, :]
            if k == 0 or k == _NK - 1:
                acc1 = acc1 + v
            else:
                acc2 = acc2 + v
        part = acc1 + 2.0 * acc2

        @pl.when(gid == _G1)
        def _():
            acc_ref[...] = part

        @pl.when(gid > _G1)
        def _():
            acc_ref[...] = acc_ref[...] + part

    @pl.when(gid == _G1 + nsteps)
    def _finish():
        ps = jnp.sum(acc_ref[...], axis=0, keepdims=True)
        tot = jnp.sum(ps, axis=1, keepdims=True)
        logpart = jnp.log(tot - jnp.float32(_NOBS))
        pij = pij_ref[...]
        out_ref[...] = pij * (jnp.log(pij) + jnp.log(den_ref[...]) + logpart)


def kernel(pij, noise_full, noise_i, noise_j, logits_weight, topic_w, topic_b, i, j):
    f32 = jnp.float32
    i32 = i.astype(jnp.int32)
    j32 = j.astype(jnp.int32)
    sti = (i32 >> 3) << 3
    ami = (-i32) & 7
    stj = (j32 >> 3) << 3
    amj = (-j32) & 7

    nsteps = _G2 // _RB
    gclamp = nsteps - 1

    def _pclamp(g):
        return (jnp.minimum(g, _G1 - 1), 0)

    def _mclamp(g):
        return (jnp.clip(g - _G1, 0, gclamp), 0)

    loss = pl.pallas_call(
        _main_body,
        grid=(_G1 + nsteps + 1,),
        in_specs=[
            pl.BlockSpec((_BM1, _NT), _pclamp),
            pl.BlockSpec((_BM1, _NT), _pclamp),
            pl.BlockSpec((_ND, _NT), lambda g: (0, 0)),
            pl.BlockSpec((_ND, 1), lambda g: (0, 0)),
            pl.BlockSpec((_NOBS, _NT), lambda g: (0, 0)),
            pl.BlockSpec((_RB * _BB2, _NT), _mclamp),
            pl.BlockSpec((_RB * _BB2, _NT), _mclamp),
            pl.BlockSpec(memory_space=pltpu.SMEM),
            pl.BlockSpec(memory_space=pltpu.SMEM),
            pl.BlockSpec(memory_space=pltpu.SMEM),
            pl.BlockSpec(memory_space=pltpu.SMEM),
            pl.BlockSpec((_ND, _NT), lambda g: (0, 0)),
            pl.BlockSpec((1, _B), lambda g: (0, 0)),
        ],
        out_specs=pl.BlockSpec((1, _B), lambda g: (0, 0)),
        out_shape=jax.ShapeDtypeStruct((1, _B), f32),
        scratch_shapes=[
            pltpu.VMEM((8, 128), f32),
            pltpu.VMEM((1, _B), f32),
            pltpu.VMEM((_RB * _BB2, _NT), f32),
            pltpu.VMEM((_RB * _BB2, _NT), f32),
            pltpu.VMEM((_NOBS, _NT), jnp.bfloat16),
            pltpu.VMEM((_NOBS, _NT), jnp.bfloat16),
        ],
        compiler_params=pltpu.CompilerParams(
            dimension_semantics=("arbitrary",)),
        name="topic_fused",
    )(logits_weight, noise_full, topic_w, topic_b.reshape(_ND, 1),
      logits_weight, noise_i, noise_j,
      sti, ami, stj, amj, topic_w, pij.reshape(1, _B))

    return loss.reshape(_B)
